# baseline (speedup 1.0000x reference)
import functools

import jax
import jax.numpy as jnp
from jax import lax
from jax.experimental import pallas as pl
from jax.experimental.pallas import tpu as pltpu
from jax.experimental.pallas import tpu_sc as plsc

NUM_EMBEDDINGS = 1000000
DIM = 64
NB = 16384
SEQ = 26
BATCH = NB * SEQ
NUM_CORES = 2
NUM_SUBCORES = 16
NW = NUM_CORES * NUM_SUBCORES
BCHUNK = 8
CROWS = BCHUNK * SEQ
RVROWS = BCHUNK * 32
B_PER_W = NB // NW
NCHUNK = B_PER_W // BCHUNK
LG = CROWS // 16

_mesh = plsc.VectorSubcoreMesh(core_axis_name="c", subcore_axis_name="s")


@functools.partial(
    pl.kernel,
    out_type=jax.ShapeDtypeStruct((NB, SEQ, DIM), jnp.float32),
    mesh=_mesh,
    scratch_types=[
        pltpu.VMEM((CROWS,), jnp.int32),
        pltpu.VMEM((CROWS,), jnp.int32),
        pltpu.VMEM((RVROWS, DIM), jnp.float32),
        pltpu.VMEM((RVROWS, DIM), jnp.float32),
        pltpu.SemaphoreType.DMA,
        pltpu.SemaphoreType.DMA,
        pltpu.SemaphoreType.DMA,
        pltpu.SemaphoreType.DMA,
        pltpu.SemaphoreType.DMA,
        pltpu.SemaphoreType.DMA,
    ],
    compiler_params=pltpu.CompilerParams(use_tc_tiling_on_sc=True),
)
def _emb_lookup(idx_hbm, table_hbm, out_hbm,
                iv0, iv1, rv0, rv1, is0, is1, gs0, gs1, os0, os1):
    IV, RV = [iv0, iv1], [rv0, rv1]
    IS, GS, OS = [is0, is1], [gs0, gs1], [os0, os1]
    wid = lax.axis_index("s") * NUM_CORES + lax.axis_index("c")
    b0 = wid * B_PER_W
    r0 = b0 * SEQ

    def idx_load(c, b):
        pltpu.async_copy(idx_hbm.at[pl.ds(r0 + c * CROWS, CROWS)], IV[b], IS[b])

    def idx_wait(b):
        pltpu.make_async_copy(idx_hbm.at[pl.ds(0, CROWS)], IV[b], IS[b]).wait()

    def gathers(b):
        for g in range(LG):
            v = IV[b][pl.ds(g * 16, 16)]
            for s in range(16):
                i = v[s]
                n = g * 16 + s
                slot = 32 * (n // SEQ) + n % SEQ
                pltpu.async_copy(
                    table_hbm.at[i >> 3, pl.ds(i & 7, 1), :],
                    RV[b].at[pl.ds(slot, 1), :],
                    GS[b],
                )

    def drain(sem, b):
        for k in range(BCHUNK):
            pltpu.make_async_copy(
                RV[b].at[pl.ds(k * 32, SEQ), :], out_hbm.at[0], sem).wait()

    def store(c, b):
        for k in range(BCHUNK):
            pltpu.async_copy(
                RV[b].at[pl.ds(k * 32, SEQ), :],
                out_hbm.at[b0 + c * BCHUNK + k],
                OS[b],
            )

    idx_load(0, 0)
    idx_wait(0)
    gathers(0)
    idx_load(1, 1)
    idx_wait(1)
    gathers(1)
    drain(GS[0], 0)
    store(0, 0)
    idx_load(2, 0)

    def outer(g, carry):
        for b in (0, 1):
            c = 2 * g + b
            idx_wait(b)
            drain(OS[b], b)
            gathers(b)
            drain(GS[1 - b], 1 - b)
            store(c - 1, 1 - b)
            idx_load(c + 1, 1 - b)
        return carry

    lax.fori_loop(1, NCHUNK // 2, outer, 0)

    idx_wait(0)
    drain(GS[1], 1)
    store(NCHUNK - 1, 1)
    drain(OS[0], 0)
    drain(OS[1], 1)


def kernel(indices, weight):
    idx = indices.astype(jnp.int32).reshape(-1)
    idx = jnp.concatenate([idx, jnp.zeros((CROWS,), jnp.int32)])
    return _emb_lookup(idx, weight.reshape(125000, 8, DIM))


# speedup vs baseline: 1.0173x; 1.0173x over previous
import functools

import jax
import jax.numpy as jnp
from jax import lax
from jax.experimental import pallas as pl
from jax.experimental.pallas import tpu as pltpu
from jax.experimental.pallas import tpu_sc as plsc

NUM_EMBEDDINGS = 1000000
DIM = 64
NB = 16384
SEQ = 26
BATCH = NB * SEQ
NUM_CORES = 2
NUM_SUBCORES = 16
NW = NUM_CORES * NUM_SUBCORES
BCHUNK = 8
CROWS = BCHUNK * SEQ
RVROWS = BCHUNK * 32
B_PER_W = NB // NW
NCHUNK = B_PER_W // BCHUNK
LG = CROWS // 16

_mesh = plsc.VectorSubcoreMesh(core_axis_name="c", subcore_axis_name="s")


@functools.partial(
    pl.kernel,
    out_type=jax.ShapeDtypeStruct((NB, SEQ, DIM), jnp.float32),
    mesh=_mesh,
    scratch_types=[
        pltpu.VMEM((CROWS,), jnp.int32),
        pltpu.VMEM((CROWS,), jnp.int32),
        pltpu.VMEM((RVROWS, DIM), jnp.float32),
        pltpu.VMEM((RVROWS, DIM), jnp.float32),
        pltpu.SemaphoreType.DMA,
        pltpu.SemaphoreType.DMA,
        pltpu.SemaphoreType.DMA,
        pltpu.SemaphoreType.DMA,
        pltpu.SemaphoreType.DMA,
        pltpu.SemaphoreType.DMA,
    ],
    compiler_params=pltpu.CompilerParams(use_tc_tiling_on_sc=True),
)
def _emb_lookup(idx_hbm, table_hbm, out_hbm,
                iv0, iv1, rv0, rv1, is0, is1, gs0, gs1, os0, os1):
    IV, RV = [iv0, iv1], [rv0, rv1]
    IS, GS, OS = [is0, is1], [gs0, gs1], [os0, os1]
    wid = lax.axis_index("s") * NUM_CORES + lax.axis_index("c")
    b0 = wid * B_PER_W
    r0 = b0 * SEQ

    def idx_load(c, b):
        pltpu.async_copy(idx_hbm.at[pl.ds(r0 + c * CROWS, CROWS)], IV[b], IS[b])

    def idx_wait(b):
        pltpu.make_async_copy(idx_hbm.at[pl.ds(0, CROWS)], IV[b], IS[b]).wait()

    def gathers(b):
        def group(g, carry):
            v = IV[b][pl.ds(g * 16, 16)]
            base = g * 16
            for s in range(16):
                i = v[s]
                n = base + s
                k = n // SEQ
                r = n % SEQ
                pltpu.async_copy(
                    table_hbm.at[i >> 3, pl.ds(i & 7, 1), :],
                    RV[b].at[pl.ds(k * 32 + r, 1), :],
                    GS[b],
                )
            return carry

        lax.fori_loop(0, LG, group, 0)

    def drain(sem, b):
        for k in range(BCHUNK):
            pltpu.make_async_copy(
                RV[b].at[pl.ds(k * 32, SEQ), :], out_hbm.at[0], sem).wait()

    def store(c, b):
        for k in range(BCHUNK):
            pltpu.async_copy(
                RV[b].at[pl.ds(k * 32, SEQ), :],
                out_hbm.at[b0 + c * BCHUNK + k],
                OS[b],
            )

    idx_load(0, 0)
    idx_wait(0)
    gathers(0)
    idx_load(1, 1)
    idx_wait(1)
    gathers(1)
    drain(GS[0], 0)
    store(0, 0)
    idx_load(2, 0)

    def outer(g, carry):
        for b in (0, 1):
            c = 2 * g + b
            idx_wait(b)
            drain(OS[b], b)
            gathers(b)
            drain(GS[1 - b], 1 - b)
            store(c - 1, 1 - b)
            idx_load(c + 1, 1 - b)
        return carry

    lax.fori_loop(1, NCHUNK // 2, outer, 0)

    idx_wait(0)
    drain(GS[1], 1)
    store(NCHUNK - 1, 1)
    drain(OS[0], 0)
    drain(OS[1], 1)


def kernel(indices, weight):
    idx = indices.astype(jnp.int32).reshape(-1)
    idx = jnp.concatenate([idx, jnp.zeros((CROWS,), jnp.int32)])
    return _emb_lookup(idx, weight.reshape(125000, 8, DIM))
